# Initial kernel scaffold; baseline (speedup 1.0000x reference)
import jax
import jax.numpy as jnp
from jax.experimental import pallas as pl


def kernel(x, edge_index, edge_attr, batch, params):
    raise NotImplementedError("write your pallas kernel here")


# speedup vs baseline: 19.8299x; 19.8299x over previous
import functools

import numpy as np
import jax
import jax.numpy as jnp
from jax import lax
from jax.experimental import pallas as pl
from jax.experimental.pallas import tpu as pltpu
from jax.experimental.pallas import tpu_sc as plsc

N, E, G, H, D, HC = 10000, 160000, 1000, 8, 32, 256
NC, NS, L = 2, 16, 16
EP = 174080
PER = EP // NS
CH1 = 544
CH2 = 128
NB = 2000
N2 = 10240
ROWS = N2 // NS
G2 = 1024
GR = G2 // NS

_P1a = np.zeros((5, 20, 10), np.float32)
_P1b = np.zeros((5, 20, 10), np.float32)
for _k in range(5):
    for _s in range(20):
        for _u in range(10):
            if _s - 2 * _u + 2 == _k:
                _P1a[_k, _s, _u] = 1.0
            if _s - 2 * _u + 1 == _k:
                _P1b[_k, _s, _u] = 1.0
_Q2 = np.zeros((3, 10, 10), np.float32)
for _k in range(3):
    for _s in range(10):
        for _t in range(10):
            if _s - _t + 1 == _k:
                _Q2[_k, _s, _t] = 1.0
_EYE8 = np.eye(H, dtype=np.float32)


def _f32(x):
    return jnp.asarray(x, jnp.float32)


def _stats_body(ea_ref, c3_ref, out_ref):
    ea = ea_ref[...]
    s = jnp.sum(ea)
    mx = jnp.max(ea)
    mn = jnp.min(ea)
    mean = s / E
    c3 = c3_ref[...]
    term = jnp.maximum(jnp.maximum(c3 * mx, c3 * mn), c3 * mean)
    out_ref[...] = jnp.concatenate(
        [term, jnp.full((3, 8), mean, jnp.float32)], axis=1)


def _ea_stats(ea2, c3):
    return pl.pallas_call(
        _stats_body,
        out_shape=jax.ShapeDtypeStruct((3, 16), jnp.float32),
        in_specs=[pl.BlockSpec((1250, 128), lambda: (0, 0)),
                  pl.BlockSpec((3, 8), lambda: (0, 0))],
        out_specs=pl.BlockSpec((3, 16), lambda: (0, 0)),
    )(ea2, c3)


def _dense_tail(xp, ssrc, sdst, i, xps_ref, t_ref, ub_ref):
    a_s = jnp.dot(xp, ssrc, preferred_element_type=jnp.float32)
    a_d = jnp.dot(xp, sdst, preferred_element_type=jnp.float32)
    xps_ref[0] = xp[:, :128]
    xps_ref[1] = xp[:, 128:]
    t_ref[...] = jnp.concatenate([a_s, a_d], axis=1)
    bm = jnp.concatenate(
        [jnp.max(a_s, axis=0, keepdims=True),
         jnp.max(a_d, axis=0, keepdims=True)], axis=1)

    @pl.when(i == 0)
    def _():
        ub_ref[...] = jnp.full((1, 16), -jnp.inf, jnp.float32)

    ub_ref[...] = jnp.maximum(ub_ref[...], bm)


def _dense1_body(x_ref, m1a_ref, m1b_ref, bveo_ref, m2_ref, b2v_ref,
                 bmat_ref, lb_ref, wt_ref, ssrc_ref, sdst_ref,
                 xps_ref, t_ref, ub_ref):
    i = pl.program_id(0)
    x = x_ref[...]
    raw = x[:, :20]
    h1a = jax.nn.relu(jnp.dot(raw, m1a_ref[...],
                              preferred_element_type=jnp.float32)
                      + bveo_ref[...])
    h1b = jax.nn.relu(jnp.dot(raw, m1b_ref[...],
                              preferred_element_type=jnp.float32)
                      + bveo_ref[...])
    hp = jnp.maximum(h1a, h1b)
    h2 = jax.nn.relu(jnp.dot(hp, m2_ref[...],
                             preferred_element_type=jnp.float32)
                     + b2v_ref[...])
    lid = jax.nn.relu(jnp.dot(h2, bmat_ref[...],
                              preferred_element_type=jnp.float32)
                      + lb_ref[...])
    h = jnp.concatenate([x[:, 20:26], lid, x[:, 26:29]], axis=1)
    xp = jnp.dot(h, wt_ref[...], preferred_element_type=jnp.float32)
    _dense_tail(xp, ssrc_ref[...], sdst_ref[...], i, xps_ref, t_ref, ub_ref)


def _dense1(x, m1a, m1b, bveo, m2, b2v, bmat, lb, wt, ssrc, sdst):
    grid = N // NB
    full = lambda r, c: pl.BlockSpec((r, c), lambda i: (0, 0))
    return pl.pallas_call(
        _dense1_body,
        grid=(grid,),
        out_shape=(jax.ShapeDtypeStruct((2, N, 128), jnp.float32),
                   jax.ShapeDtypeStruct((N2, 16), jnp.float32),
                   jax.ShapeDtypeStruct((1, 16), jnp.float32)),
        in_specs=[pl.BlockSpec((NB, 29), lambda i: (i, 0)),
                  full(20, 160), full(20, 160), full(1, 160),
                  full(160, 320), full(1, 320), full(320, 16), full(1, 16),
                  full(25, HC), full(HC, 8), full(HC, 8)],
        out_specs=(pl.BlockSpec((2, NB, 128), lambda i: (0, i, 0)),
                   pl.BlockSpec((NB, 16), lambda i: (i, 0)),
                   pl.BlockSpec((1, 16), lambda i: (0, 0))),
    )(x, m1a, m1b, bveo, m2, b2v, bmat, lb, wt, ssrc, sdst)


def _densen_body(oe_ref, bias_ref, wt_ref, ssrc_ref, sdst_ref,
                 xps_ref, t_ref, ub_ref):
    i = pl.program_id(0)
    h = jax.nn.relu(jnp.concatenate([oe_ref[0], oe_ref[1]], axis=1)
                    + bias_ref[...])
    xp = jnp.dot(h, wt_ref[...], preferred_element_type=jnp.float32)
    _dense_tail(xp, ssrc_ref[...], sdst_ref[...], i, xps_ref, t_ref, ub_ref)


def _densen(oe, bias, wt, ssrc, sdst):
    grid = N // NB
    full = lambda r, c: pl.BlockSpec((r, c), lambda i: (0, 0))
    return pl.pallas_call(
        _densen_body,
        grid=(grid,),
        out_shape=(jax.ShapeDtypeStruct((2, N, 128), jnp.float32),
                   jax.ShapeDtypeStruct((N2, 16), jnp.float32),
                   jax.ShapeDtypeStruct((1, 16), jnp.float32)),
        in_specs=[pl.BlockSpec((2, NB, 128), lambda i: (0, i, 0)),
                  full(1, HC), full(HC, HC), full(HC, 8), full(HC, 8)],
        out_specs=(pl.BlockSpec((2, NB, 128), lambda i: (0, i, 0)),
                   pl.BlockSpec((NB, 16), lambda i: (i, 0)),
                   pl.BlockSpec((1, 16), lambda i: (0, 0))),
    )(oe, bias, wt, ssrc, sdst)


def _post3_body(oe_ref, bias_ref, h3s_ref):
    h = jax.nn.relu(jnp.concatenate([oe_ref[0], oe_ref[1]], axis=1)
                    + bias_ref[...])
    h3s_ref[0] = h[:, :128]
    h3s_ref[1] = h[:, 128:]


def _post3(oe, bias):
    nb2 = 2048
    grid = N2 // nb2
    return pl.pallas_call(
        _post3_body,
        grid=(grid,),
        out_shape=jax.ShapeDtypeStruct((2, N2, 128), jnp.float32),
        in_specs=[pl.BlockSpec((2, nb2, 128), lambda i: (0, i, 0)),
                  pl.BlockSpec((1, HC), lambda i: (0, 0))],
        out_specs=pl.BlockSpec((2, nb2, 128), lambda i: (0, i, 0)),
    )(oe, bias)


def _head_body(s_ref, cnt_ref, w1_ref, b1_ref, w2_ref, b2_ref, out_ref):
    pooled = jnp.concatenate([s_ref[0], s_ref[1]], axis=1)
    pooled = pooled / jnp.maximum(cnt_ref[...][:, 0:1], 1.0)
    ch = jax.nn.relu(jnp.dot(pooled, w1_ref[...],
                             preferred_element_type=jnp.float32)
                     + b1_ref[...])
    out_ref[...] = jnp.dot(ch, w2_ref[...],
                           preferred_element_type=jnp.float32) + b2_ref[...]


def _head(sums, cnt, w1t, b1, w2t, b2):
    full = lambda r, c: pl.BlockSpec((r, c), lambda: (0, 0))
    return pl.pallas_call(
        _head_body,
        out_shape=jax.ShapeDtypeStruct((G, 1), jnp.float32),
        in_specs=[pl.BlockSpec((2, G, 128), lambda: (0, 0, 0)),
                  full(G, 8), full(HC, 128), full(1, 128),
                  full(128, 1), full(1, 1)],
        out_specs=full(G, 1),
    )(sums, cnt, w1t, b1, w2t, b2)


_MESH = plsc.VectorSubcoreMesh(core_axis_name="c", subcore_axis_name="s",
                               num_cores=NC, num_subcores=NS)


def _splat(ref, val):
    if isinstance(val, int):
        idx = jnp.full((L,), val, jnp.int32)
    else:
        idx = jnp.broadcast_to(val.astype(jnp.int32), (L,))
    return plsc.load_gather(ref, [idx])


def _gat_sc_body(t_h, xps_h, src_h, dst_h, ea_h, mk_h, cv_h, ub_h, st_h,
                 z8_h, z128_h, oe_h,
                 cv, ubv, sv, ubsum, den_sp, out_sp, sem, sem2, sem3):
    cid = lax.axis_index("c")
    sid = lax.axis_index("s")
    iota = lax.iota(jnp.int32, L)

    pltpu.sync_copy(cv_h, cv)
    pltpu.sync_copy(ub_h.at[0], ubv)
    pltpu.sync_copy(st_h.at[0], sv)
    im = lax.bitwise_and(iota, 7)
    ubsum[...] = (plsc.load_gather(ubv, [im])
                  + plsc.load_gather(ubv, [im + 8])
                  + plsc.load_gather(sv, [im]))

    pltpu.sync_copy(z8_h.at[pl.ds(sid * ROWS, ROWS)],
                    den_sp.at[pl.ds(sid * ROWS, ROWS)])
    pltpu.sync_copy(z128_h.at[pl.ds(sid * ROWS, ROWS)],
                    out_sp.at[pl.ds(sid * ROWS, ROWS)])
    plsc.subcore_barrier()

    base = sid * PER

    def pass1(idx1s, idx1d, g1a, g1b, ea1, mk1, ebuf):
        def chunk1(k, carry):
            off = base + k * CH1
            pltpu.sync_copy(src_h.at[pl.ds(off, CH1)], idx1s)
            pltpu.sync_copy(dst_h.at[pl.ds(off, CH1)], idx1d)
            pltpu.async_copy(t_h.at[idx1s], g1a, sem).wait()
            pltpu.async_copy(t_h.at[idx1d], g1b, sem2).wait()
            pltpu.sync_copy(ea_h.at[pl.ds(off, CH1)], ea1)
            pltpu.sync_copy(mk_h.at[pl.ds(off, CH1)], mk1)

            def grp(r, c2):
                rows16 = r * L + iota
                eav = ea1[pl.ds(r * L, L)]
                mkv = mk1[pl.ds(r * L, L)]
                for h in range(H):
                    hf = jnp.full((L,), h, jnp.int32)
                    vas = plsc.load_gather(g1a, [rows16, hf])
                    vad = plsc.load_gather(g1b, [rows16, hf + 8])
                    pre = vas + vad + eav * _splat(cv, h)
                    al = jnp.maximum(pre, 0.2 * pre)
                    ev = jnp.exp(al - _splat(ubsum, h)) * mkv
                    plsc.store_scatter(ebuf, [rows16, hf], ev)
                return c2

            lax.fori_loop(0, CH1 // L, grp, 0)
            pltpu.sync_copy(ebuf, den_sp.at[idx1d], add=True)
            return carry

        lax.fori_loop(0, PER // CH1, chunk1, 0)

    pl.run_scoped(
        pass1,
        pltpu.VMEM((CH1,), jnp.int32), pltpu.VMEM((CH1,), jnp.int32),
        pltpu.VMEM((CH1, 16), jnp.float32), pltpu.VMEM((CH1, 16), jnp.float32),
        pltpu.VMEM((CH1,), jnp.float32), pltpu.VMEM((CH1,), jnp.float32),
        pltpu.VMEM((CH1, 8), jnp.float32),
    )
    plsc.subcore_barrier()

    coff = cid * N

    def pass2(idx2s, idx2d, xg, g2a, g2b, ea2, mk2, dg, attb):
        def chunk2(k, carry):
            off = base + k * CH2
            pltpu.sync_copy(src_h.at[pl.ds(off, CH2)], idx2s)
            pltpu.sync_copy(dst_h.at[pl.ds(off, CH2)], idx2d)
            pltpu.async_copy(t_h.at[idx2s], g2a, sem).wait()
            pltpu.async_copy(t_h.at[idx2d], g2b, sem2).wait()
            pltpu.async_copy(den_sp.at[idx2d], dg, sem3).wait()
            pltpu.sync_copy(ea_h.at[pl.ds(off, CH2)], ea2)
            pltpu.sync_copy(mk_h.at[pl.ds(off, CH2)], mk2)

            def sh(r, c2):
                v = idx2s[pl.ds(r * L, L)]
                idx2s[pl.ds(r * L, L)] = v + coff
                return c2

            lax.fori_loop(0, CH2 // L, sh, 0)
            pltpu.async_copy(xps_h.at[idx2s], xg, sem).wait()

            def grp2(r, c2):
                rows16 = r * L + iota
                eav = ea2[pl.ds(r * L, L)]
                mkv = mk2[pl.ds(r * L, L)]
                for hl in range(4):
                    hh = cid * 4 + hl
                    hf = jnp.broadcast_to(hh, (L,)).astype(jnp.int32)
                    vas = plsc.load_gather(g2a, [rows16, hf])
                    vad = plsc.load_gather(g2b, [rows16, hf + 8])
                    pre = vas + vad + eav * _splat(cv, hh)
                    al = jnp.maximum(pre, 0.2 * pre)
                    ev = jnp.exp(al - _splat(ubsum, hh)) * mkv
                    dv = plsc.load_gather(dg, [rows16, hf])
                    attb[hl, pl.ds(r * L, L)] = ev / (dv + 1e-16)
                return c2

            lax.fori_loop(0, CH2 // L, grp2, 0)

            def edge2(j, c2):
                for hl in range(4):
                    b = plsc.load_gather(
                        attb, [jnp.full((L,), hl, jnp.int32),
                               jnp.broadcast_to(j, (L,)).astype(jnp.int32)])
                    for q in range(2):
                        col = hl * 32 + q * L
                        xg[j, pl.ds(col, L)] = xg[j, pl.ds(col, L)] * b
                return c2

            lax.fori_loop(0, CH2, edge2, 0)
            pltpu.sync_copy(xg, out_sp.at[idx2d], add=True)
            return carry

        lax.fori_loop(0, PER // CH2, chunk2, 0)

    pl.run_scoped(
        pass2,
        pltpu.VMEM((CH2,), jnp.int32), pltpu.VMEM((CH2,), jnp.int32),
        pltpu.VMEM((CH2, 128), jnp.float32),
        pltpu.VMEM((CH2, 16), jnp.float32), pltpu.VMEM((CH2, 16), jnp.float32),
        pltpu.VMEM((CH2,), jnp.float32), pltpu.VMEM((CH2,), jnp.float32),
        pltpu.VMEM((CH2, 8), jnp.float32), pltpu.VMEM((4, CH2), jnp.float32),
    )
    plsc.subcore_barrier()

    pltpu.sync_copy(out_sp.at[pl.ds(sid * ROWS, ROWS)],
                    oe_h.at[pl.ds(cid * N2 + sid * ROWS, ROWS)])


_gat_sc = functools.partial(
    pl.kernel,
    _gat_sc_body,
    out_type=jax.ShapeDtypeStruct((NC * N2, 128), jnp.float32),
    mesh=_MESH,
    compiler_params=pltpu.CompilerParams(needs_layout_passes=False, use_tc_tiling_on_sc=False),
    scratch_types=[
        pltpu.VMEM((8,), jnp.float32), pltpu.VMEM((16,), jnp.float32),
        pltpu.VMEM((16,), jnp.float32), pltpu.VMEM((16,), jnp.float32),
        pltpu.VMEM_SHARED((N2, 8), jnp.float32),
        pltpu.VMEM_SHARED((N2, 128), jnp.float32),
        pltpu.SemaphoreType.DMA, pltpu.SemaphoreType.DMA,
        pltpu.SemaphoreType.DMA,
    ],
)()


def _pool_sc_body(h3s_h, batch_h, ones_h, zg128_h, zg8_h, sums_h, cnt_h,
                  buf, bidx, onev, acc_sp, cnt_sp, sem):
    cid = lax.axis_index("c")
    sid = lax.axis_index("s")

    pltpu.sync_copy(zg128_h.at[pl.ds(sid * GR, GR)],
                    acc_sp.at[pl.ds(sid * GR, GR)])
    pltpu.sync_copy(zg8_h.at[pl.ds(sid * GR, GR)],
                    cnt_sp.at[pl.ds(sid * GR, GR)])
    plsc.subcore_barrier()
    base = sid * ROWS
    pltpu.sync_copy(batch_h.at[pl.ds(base, ROWS)], bidx)
    pltpu.sync_copy(h3s_h.at[pl.ds(cid * N2 + base, ROWS)], buf)
    pltpu.sync_copy(ones_h, onev)
    pltpu.sync_copy(buf, acc_sp.at[bidx], add=True)
    pltpu.sync_copy(onev, cnt_sp.at[bidx], add=True)
    plsc.subcore_barrier()

    pltpu.sync_copy(acc_sp.at[pl.ds(sid * GR, GR)],
                    sums_h.at[pl.ds(cid * G2 + sid * GR, GR)])

    @pl.when(cid == 0)
    def _():
        pltpu.sync_copy(cnt_sp.at[pl.ds(sid * GR, GR)],
                        cnt_h.at[pl.ds(sid * GR, GR)])


_pool_sc = functools.partial(
    pl.kernel,
    _pool_sc_body,
    out_type=(jax.ShapeDtypeStruct((NC * G2, 128), jnp.float32),
              jax.ShapeDtypeStruct((G2, 8), jnp.float32)),
    mesh=_MESH,
    compiler_params=pltpu.CompilerParams(needs_layout_passes=False, use_tc_tiling_on_sc=False),
    scratch_types=[
        pltpu.VMEM((ROWS, 128), jnp.float32),
        pltpu.VMEM((ROWS,), jnp.int32),
        pltpu.VMEM((ROWS, 8), jnp.float32),
        pltpu.VMEM_SHARED((G2, 128), jnp.float32),
        pltpu.VMEM_SHARED((G2, 8), jnp.float32),
        pltpu.SemaphoreType.DMA,
    ],
)()


def kernel(x, edge_index, edge_attr, batch, params):
    p = params

    m1a = jnp.einsum('ck,ksu->scu', p['c1w'][:, 0, :], _P1a).reshape(20, 160)
    m1b = jnp.einsum('ck,ksu->scu', p['c1w'][:, 0, :], _P1b).reshape(20, 160)
    bveo = jnp.repeat(p['c1b'], 10)[None, :]
    m2 = jnp.einsum('oik,kst->isot', p['c2w'], _Q2).reshape(160, 320)
    b2v = jnp.repeat(p['c2b'], 10)[None, :]
    bmat = (jnp.repeat(p['lw'].T, 10, axis=0).reshape(32, 10, 16)
            .reshape(320, 16) / 10.0)
    lb = p['lb'][None, :]

    gps = (p['gat1'], p['gat2'], p['gat3'])
    cs, ssrcs, sdsts, wts = [], [], [], []
    for gp in gps:
        cs.append(jnp.sum(gp['W_edge'][:, 0].reshape(H, D)
                          * gp['att_edge'][0], axis=1))
        ssrcs.append((gp['att_src'][0][:, :, None]
                      * _EYE8[:, None, :]).reshape(HC, H))
        sdsts.append((gp['att_dst'][0][:, :, None]
                      * _EYE8[:, None, :]).reshape(HC, H))
        wts.append(gp['W'].T)
    c3 = jnp.stack(cs)

    ea2 = edge_attr[:, 0].reshape(1250, 128)
    stats = _ea_stats(ea2, c3)
    eamean = stats[0, 8]

    pad = EP - E - N
    srcx = jnp.concatenate([edge_index[0], jnp.arange(N, dtype=jnp.int32),
                            jnp.zeros((pad,), jnp.int32)])
    dstx = jnp.concatenate([edge_index[1], jnp.arange(N, dtype=jnp.int32),
                            jnp.zeros((pad,), jnp.int32)])
    ea_ext = jnp.concatenate([edge_attr[:, 0],
                              jnp.full((N,), eamean, jnp.float32),
                              jnp.zeros((pad,), jnp.float32)])
    maskx = jnp.concatenate([jnp.ones((E + N,), jnp.float32),
                             jnp.zeros((pad,), jnp.float32)])
    z8 = jnp.zeros((N2, 8), jnp.float32)
    z128 = jnp.zeros((N2, 128), jnp.float32)

    xps, t, ub = _dense1(x, m1a, m1b, bveo, m2, b2v, bmat, lb,
                         wts[0], ssrcs[0], sdsts[0])
    for li in range(3):
        oe = _gat_sc(t, xps.reshape(NC * N, 128), srcx, dstx, ea_ext, maskx,
                     cs[li], ub, stats[li:li + 1], z8, z128)
        oe = oe.reshape(NC, N2, 128)
        bias = gps[li]['bias'][None, :]
        if li < 2:
            xps, t, ub = _densen(oe, bias, wts[li + 1],
                                 ssrcs[li + 1], sdsts[li + 1])
        else:
            h3s = _post3(oe, bias)

    ones8 = jnp.ones((ROWS, 8), jnp.float32)
    zg128 = jnp.zeros((G2, 128), jnp.float32)
    zg8 = jnp.zeros((G2, 8), jnp.float32)
    batch_p = jnp.concatenate([batch, jnp.full((N2 - N,), G, jnp.int32)])
    sums, cnt = _pool_sc(h3s.reshape(NC * N2, 128), batch_p, ones8,
                         zg128, zg8)

    out = _head(sums.reshape(NC, G2, 128)[:, :G], cnt[:G],
                p['fc1w'].T, p['fc1b'][None, :],
                p['fc2w'].T, p['fc2b'][None, :])
    return out
